# baseline (speedup 1.0000x reference)
import functools

import jax
import jax.numpy as jnp
from jax import lax
from jax.experimental import pallas as pl
from jax.experimental.pallas import tpu as pltpu
from jax.experimental.pallas import tpu_sc as plsc

NUM_INPUTS = 32
BATCH = 16384
NC = 2
NS = 16
NW = NC * NS
BPW = BATCH // NW
CHUNK = 128
NCHUNK = BPW // CHUNK
NGRP = BPW // 16


def _sc_body(xt_ref, ids_ref, wb_ref, tab_ref, out_ref,
             idx_v, xt_v, rows_v, out_v, wb_v, sem):
    wid = lax.axis_index("s") * NC + lax.axis_index("c")

    pltpu.sync_copy(ids_ref.at[wid], idx_v)
    pltpu.sync_copy(xt_ref.at[wid], xt_v)
    pltpu.sync_copy(wb_ref, wb_v)

    copies = []
    for k in range(NCHUNK):
        copies.append(pltpu.async_copy(
            tab_ref.at[idx_v.at[k]],
            rows_v.at[pl.ds(k * CHUNK, CHUNK)],
            sem))
    for c in copies:
        c.wait()

    lanes = lax.iota(jnp.int32, 16)

    def group(g, _):
        base = g * 16
        rowidx = base + lanes
        acc = jnp.zeros((16,), jnp.float32)
        for j in range(NUM_INPUTS):
            xv = xt_v[j, pl.ds(base, 16)]
            wv = plsc.load_gather(rows_v,
                                  [rowidx, jnp.full((16,), j, jnp.int32)])
            acc = acc + xv * (wv + wb_v[j])
        out_v[pl.ds(base, 16)] = acc
        return 0

    lax.fori_loop(0, NGRP, group, 0)

    pltpu.sync_copy(out_v, out_ref.at[wid])


@jax.jit
def _run(XT3, ids3, wb, W_random):
    mesh = plsc.VectorSubcoreMesh(core_axis_name="c", subcore_axis_name="s")
    f = functools.partial(
        pl.kernel,
        out_type=jax.ShapeDtypeStruct((NW, BPW), jnp.float32),
        mesh=mesh,
        compiler_params=pltpu.CompilerParams(needs_layout_passes=False,
                                             use_tc_tiling_on_sc=False),
        scratch_types=[
            pltpu.VMEM((NCHUNK, CHUNK), jnp.int32),
            pltpu.VMEM((NUM_INPUTS, BPW), jnp.float32),
            pltpu.VMEM((BPW, NUM_INPUTS), jnp.float32),
            pltpu.VMEM((BPW,), jnp.float32),
            pltpu.VMEM((NUM_INPUTS, 16), jnp.float32),
            pltpu.SemaphoreType.DMA,
        ],
    )(_sc_body)
    return f(XT3, ids3, wb, W_random)


def kernel(X, ids, W_weight, W_random):
    XT3 = jnp.transpose(X.reshape(NW, BPW, NUM_INPUTS), (0, 2, 1))
    ids3 = ids.astype(jnp.int32).reshape(NW, NCHUNK, CHUNK)
    wb = jnp.broadcast_to(W_weight.reshape(NUM_INPUTS, 1), (NUM_INPUTS, 16))
    out = _run(XT3, ids3, wb, W_random)
    return out.reshape(BATCH)
